# Initial kernel scaffold; baseline (speedup 1.0000x reference)
import jax
import jax.numpy as jnp
from jax.experimental import pallas as pl


def kernel(world_pos, mesh_pos, prev_world_pos, phi, prev_phi, swelling_phi, swelling_phi_rate, swelling_phi_rate_prev, node_type, mat_param, edge_index, params):
    raise NotImplementedError("write your pallas kernel here")


# speedup vs baseline: 3.9453x; 3.9453x over previous
import functools

import jax
import jax.numpy as jnp
from jax import lax
from jax.experimental import pallas as pl
from jax.experimental.pallas import tpu as pltpu
from jax.experimental.pallas import tpu_sc as plsc

N = 10000
E = 320000
H = 128

NC = 2
NS = 16
NW = NC * NS
SC_B = 80
PER_W = E // NW
SC_ITERS = PER_W // SC_B
ROW_A = 624
ROW_B = N - 15 * ROW_A

_mesh = plsc.VectorSubcoreMesh(core_axis_name="c", subcore_axis_name="s")


def _gather2(t1, t2, sidx, ridx):
    D = t1.shape[1]
    out = jax.ShapeDtypeStruct((E, D), jnp.float32)

    @functools.partial(
        pl.kernel,
        out_type=(out, out),
        mesh=_mesh,
        scratch_types=[
            pltpu.VMEM((SC_B,), jnp.int32),
            pltpu.VMEM((SC_B,), jnp.int32),
            pltpu.VMEM((SC_B, D), jnp.float32),
            pltpu.VMEM((SC_B, D), jnp.float32),
            pltpu.SemaphoreType.DMA,
            pltpu.SemaphoreType.DMA,
        ],
    )
    def k(t1_h, t2_h, s_h, r_h, o1_h, o2_h, si, ri, r1, r2, sm1, sm2):
        wid = lax.axis_index("s") * NC + lax.axis_index("c")

        def body(j, carry):
            base = wid * PER_W + j * SC_B
            pltpu.sync_copy(s_h.at[pl.ds(base, SC_B)], si)
            pltpu.sync_copy(r_h.at[pl.ds(base, SC_B)], ri)
            c1 = pltpu.async_copy(t1_h.at[si], r1, sm1)
            c2 = pltpu.async_copy(t2_h.at[ri], r2, sm2)
            c1.wait()
            c2.wait()
            pltpu.sync_copy(r1, o1_h.at[pl.ds(base, SC_B)])
            pltpu.sync_copy(r2, o2_h.at[pl.ds(base, SC_B)])
            return carry

        lax.fori_loop(0, SC_ITERS, body, 0)

    return k(t1, t2, sidx, ridx)


HN = N // NC
TRASH = HN
ACC_R = HN + 8
PER_S = E // NS
S_ITERS = PER_S // SC_B
WB_A = 312
WB_B = HN - 15 * WB_A


def _scatter_add(vals, ridx):
    @functools.partial(
        pl.kernel,
        out_type=jax.ShapeDtypeStruct((N, H), jnp.float32),
        mesh=_mesh,
        scratch_types=[
            pltpu.VMEM((SC_B,), jnp.int32),
            pltpu.VMEM((SC_B, H), jnp.float32),
            pltpu.VMEM((WB_B, H), jnp.float32),
            pltpu.VMEM_SHARED((ACC_R, H), jnp.float32),
            pltpu.SemaphoreType.DMA,
        ],
    )
    def k(v_h, r_h, o_h, idxv, rows, zbuf, acc, sem):
        c = lax.axis_index("c")
        s = lax.axis_index("s")
        lo = c * HN

        def zrow(i, carry):
            def zcol(j, cc):
                zbuf[i, pl.ds(j * 16, 16)] = jnp.zeros((16,), jnp.float32)
                return cc
            return lax.fori_loop(0, H // 16, zcol, carry)

        lax.fori_loop(0, WB_B, zrow, 0)

        @pl.when(s < 15)
        def _():
            pltpu.sync_copy(zbuf.at[pl.ds(0, WB_A)],
                            acc.at[pl.ds(s * WB_A, WB_A)])

        @pl.when(s == 15)
        def _():
            pltpu.sync_copy(zbuf, acc.at[pl.ds(15 * WB_A, WB_B)])

        plsc.subcore_barrier()

        def body(j, carry):
            base = s * PER_S + j * SC_B
            pltpu.sync_copy(r_h.at[pl.ds(base, SC_B)], idxv)
            pltpu.sync_copy(v_h.at[pl.ds(base, SC_B)], rows)
            for t in range(SC_B // 16):
                v = idxv[pl.ds(t * 16, 16)] - lo
                ok = (v >= 0) & (v < HN)
                idxv[pl.ds(t * 16, 16)] = jnp.where(ok, v, TRASH)
            pltpu.sync_copy(rows, acc.at[idxv], add=True)
            return carry

        lax.fori_loop(0, S_ITERS, body, 0)
        plsc.subcore_barrier()

        @pl.when(s < 15)
        def _():
            pltpu.sync_copy(acc.at[pl.ds(s * WB_A, WB_A)],
                            o_h.at[pl.ds(lo + s * WB_A, WB_A)])

        @pl.when(s == 15)
        def _():
            pltpu.sync_copy(acc.at[pl.ds(15 * WB_A, WB_B)],
                            o_h.at[pl.ds(lo + 15 * WB_A, WB_B)])

    return k(vals, ridx)


def _ln(h, g, b):
    m = jnp.mean(h, axis=-1, keepdims=True)
    v = jnp.mean((h - m) * (h - m), axis=-1, keepdims=True)
    return (h - m) * lax.rsqrt(v + 1e-5) * g + b


def _dot(a, b):
    return jnp.dot(a, b, preferred_element_type=jnp.float32)


def _full(shape):
    return pl.BlockSpec(shape, lambda i: (0,) * len(shape))


def _rows(blk, d):
    return pl.BlockSpec((blk, d), lambda i: (i, 0))


N_BLK = 2000
E_BLK = 2560


def _tc_enc_node(x16, w0, b0, w1, b1, g, bl, wa, wb):
    def body(x_r, w0_r, b0_r, w1_r, b1_r, g_r, bl_r, wa_r, wb_r,
             xh_r, a1_r, a2_r):
        h = jnp.maximum(_dot(x_r[...], w0_r[...]) + b0_r[...], 0.0)
        xh = _ln(_dot(h, w1_r[...]) + b1_r[...], g_r[...], bl_r[...])
        xh_r[...] = xh
        a1_r[...] = _dot(xh, wa_r[...])
        a2_r[...] = _dot(xh, wb_r[...])

    o = jax.ShapeDtypeStruct((N, H), jnp.float32)
    return pl.pallas_call(
        body,
        grid=(N // N_BLK,),
        in_specs=[_rows(N_BLK, 16), _full((16, H)), _full((1, H)),
                  _full((H, H)), _full((1, H)), _full((1, H)), _full((1, H)),
                  _full((H, H)), _full((H, H))],
        out_specs=[_rows(N_BLK, H)] * 3,
        out_shape=[o, o, o],
    )(x16, w0, b0, w1, b1, g, bl, wa, wb)


def _tc_enc_edge(gs, gr, wlin, wd, wdw, b0, w1, b1, g, bl):
    def body(gs_r, gr_r, wlin_r, wd_r, wdw_r, b0_r, w1_r, b1_r, g_r, bl_r,
             eh_r):
        rel = gs_r[...] - gr_r[...]
        d2 = rel[:, 0:1] * rel[:, 0:1] + rel[:, 1:2] * rel[:, 1:2]
        dw2 = rel[:, 2:3] * rel[:, 2:3] + rel[:, 3:4] * rel[:, 3:4]
        pre = (_dot(rel, wlin_r[...])
               + jnp.sqrt(d2) * wd_r[...]
               + jnp.sqrt(dw2) * wdw_r[...] + b0_r[...])
        h = jnp.maximum(pre, 0.0)
        eh_r[...] = _ln(_dot(h, w1_r[...]) + b1_r[...], g_r[...], bl_r[...])

    return pl.pallas_call(
        body,
        grid=(E // E_BLK,),
        in_specs=[_rows(E_BLK, H), _rows(E_BLK, H), _full((H, H)),
                  _full((1, H)), _full((1, H)), _full((1, H)),
                  _full((H, H)), _full((1, H)), _full((1, H)), _full((1, H))],
        out_specs=[_rows(E_BLK, H)],
        out_shape=[jax.ShapeDtypeStruct((E, H), jnp.float32)],
    )(gs, gr, wlin, wd, wdw, b0, w1, b1, g, bl)[0]


def _tc_edge_step(g1, g2, eh, w3, b0, w1, b1, g, bl):
    def body(g1_r, g2_r, eh_r, w3_r, b0_r, w1_r, b1_r, g_r, bl_r,
             ne_r, en_r):
        eh_v = eh_r[...]
        t = jnp.maximum(g1_r[...] + g2_r[...] + _dot(eh_v, w3_r[...])
                        + b0_r[...], 0.0)
        t = jnp.maximum(_dot(t, w1_r[...]) + b1_r[...], 0.0)
        ne = _ln(t, g_r[...], bl_r[...])
        ne_r[...] = ne
        en_r[...] = ne + eh_v

    o = jax.ShapeDtypeStruct((E, H), jnp.float32)
    return pl.pallas_call(
        body,
        grid=(E // E_BLK,),
        in_specs=[_rows(E_BLK, H)] * 3 + [_full((H, H)), _full((1, H)),
                  _full((H, H)), _full((1, H)), _full((1, H)), _full((1, H))],
        out_specs=[_rows(E_BLK, H)] * 2,
        out_shape=[o, o],
    )(g1, g2, eh, w3, b0, w1, b1, g, bl)


def _tc_node_step(xh, aggr, w0a, w0b, b0, w1, b1, g, bl, wa, wb):
    def body(xh_r, ag_r, w0a_r, w0b_r, b0_r, w1_r, b1_r, g_r, bl_r,
             wa_r, wb_r, xn_r, a1_r, a2_r):
        xh_v = xh_r[...]
        t = jnp.maximum(_dot(xh_v, w0a_r[...]) + _dot(ag_r[...], w0b_r[...])
                        + b0_r[...], 0.0)
        t = jnp.maximum(_dot(t, w1_r[...]) + b1_r[...], 0.0)
        xn = _ln(t, g_r[...], bl_r[...]) + xh_v
        xn_r[...] = xn
        a1_r[...] = _dot(xn, wa_r[...])
        a2_r[...] = _dot(xn, wb_r[...])

    o = jax.ShapeDtypeStruct((N, H), jnp.float32)
    return pl.pallas_call(
        body,
        grid=(N // N_BLK,),
        in_specs=[_rows(N_BLK, H)] * 2 + [_full((H, H)), _full((H, H)),
                  _full((1, H)), _full((H, H)), _full((1, H)), _full((1, H)),
                  _full((1, H)), _full((H, H)), _full((H, H))],
        out_specs=[_rows(N_BLK, H)] * 3,
        out_shape=[o, o, o],
    )(xh, aggr, w0a, w0b, b0, w1, b1, g, bl, wa, wb)


def _tc_node_last(xh, aggr, w0a, w0b, b0, w1, b1, g, bl,
                  dw0, dbw0, wdw8, dp0, dbp0, wdp8, bd8):
    def body(xh_r, ag_r, w0a_r, w0b_r, b0_r, w1_r, b1_r, g_r, bl_r,
             dw0_r, dbw0_r, wdw8_r, dp0_r, dbp0_r, wdp8_r, bd8_r, o_r):
        xh_v = xh_r[...]
        t = jnp.maximum(_dot(xh_v, w0a_r[...]) + _dot(ag_r[...], w0b_r[...])
                        + b0_r[...], 0.0)
        t = jnp.maximum(_dot(t, w1_r[...]) + b1_r[...], 0.0)
        xn = _ln(t, g_r[...], bl_r[...]) + xh_v
        d1 = jnp.maximum(_dot(xn, dw0_r[...]) + dbw0_r[...], 0.0)
        d2 = jnp.maximum(_dot(xn, dp0_r[...]) + dbp0_r[...], 0.0)
        o_r[...] = _dot(d1, wdw8_r[...]) + _dot(d2, wdp8_r[...]) + bd8_r[...]

    return pl.pallas_call(
        body,
        grid=(N // N_BLK,),
        in_specs=[_rows(N_BLK, H)] * 2 + [_full((H, H)), _full((H, H)),
                  _full((1, H)), _full((H, H)), _full((1, H)), _full((1, H)),
                  _full((1, H)), _full((H, H)), _full((1, H)), _full((H, 8)),
                  _full((H, H)), _full((1, H)), _full((H, 8)), _full((1, 8))],
        out_specs=[_rows(N_BLK, 8)],
        out_shape=[jax.ShapeDtypeStruct((N, 8), jnp.float32)],
    )(xh, aggr, w0a, w0b, b0, w1, b1, g, bl,
      dw0, dbw0, wdw8, dp0, dbp0, wdp8, bd8)[0]


def kernel(world_pos, mesh_pos, prev_world_pos, phi, prev_phi, swelling_phi,
           swelling_phi_rate, swelling_phi_rate_prev, node_type, mat_param,
           edge_index, params):
    f32 = jnp.float32
    senders = edge_index[0].astype(jnp.int32)
    receivers = edge_index[1].astype(jnp.int32)

    x16 = jnp.concatenate(
        [phi, prev_phi, swelling_phi, swelling_phi_rate,
         swelling_phi_rate_prev, node_type,
         jnp.zeros((N, 2), f32)], axis=1)
    ne0w = params["ne0"]["w"]
    w0p = jnp.concatenate(
        [(ne0w[0] + ne0w[1])[None], (-ne0w[1])[None], ne0w[2:],
         jnp.zeros((2, H), f32)], axis=0)

    P = jnp.concatenate([mesh_pos, world_pos, phi, jnp.zeros((N, H - 5), f32)],
                        axis=1)
    ee0w = params["ee0"]["w"]
    wlin = jnp.concatenate([ee0w[0:2], ee0w[3:5], ee0w[6:7],
                            jnp.zeros((H - 5, H), f32)], axis=0)
    wd = ee0w[2][None]
    wdw = ee0w[5][None]

    def r1(v):
        return v[None]

    pr0 = params["procs"][0]
    x_h, a1, a2 = _tc_enc_node(
        x16, w0p, r1(params["ne0"]["b"]), params["ne1"]["w"],
        r1(params["ne1"]["b"]), r1(params["ne_ln"]["g"]),
        r1(params["ne_ln"]["b"]),
        pr0["e0"]["w"][0:H], pr0["e0"]["w"][H:2 * H])

    gs, gr = _gather2(P, P, senders, receivers)
    e_h = _tc_enc_edge(
        gs, gr, wlin, wd, wdw, r1(params["ee0"]["b"]), params["ee1"]["w"],
        r1(params["ee1"]["b"]), r1(params["ee_ln"]["g"]),
        r1(params["ee_ln"]["b"]))

    dec = None
    for k in range(3):
        pr = params["procs"][k]
        g1, g2 = _gather2(a1, a2, senders, receivers)
        new_e, e_h = _tc_edge_step(
            g1, g2, e_h, pr["e0"]["w"][2 * H:3 * H], r1(pr["e0"]["b"]),
            pr["e1"]["w"], r1(pr["e1"]["b"]), r1(pr["e_ln"]["g"]),
            r1(pr["e_ln"]["b"]))
        aggr = _scatter_add(new_e, receivers)
        nargs = (x_h, aggr, pr["n0"]["w"][0:H], pr["n0"]["w"][H:2 * H],
                 r1(pr["n0"]["b"]), pr["n1"]["w"], r1(pr["n1"]["b"]),
                 r1(pr["n_ln"]["g"]), r1(pr["n_ln"]["b"]))
        if k < 2:
            prn = params["procs"][k + 1]
            x_h, a1, a2 = _tc_node_step(
                *nargs, prn["e0"]["w"][0:H], prn["e0"]["w"][H:2 * H])
        else:
            wdw8 = jnp.zeros((H, 8), f32).at[:, 0:2].set(params["dw1"]["w"])
            wdp8 = jnp.zeros((H, 8), f32).at[:, 2:3].set(params["dp1"]["w"])
            bd8 = jnp.zeros((1, 8), f32).at[0, 0:2].set(
                params["dw1"]["b"]).at[0, 2].set(params["dp1"]["b"][0])
            dec = _tc_node_last(
                *nargs, params["dw0"]["w"], r1(params["dw0"]["b"]), wdw8,
                params["dp0"]["w"], r1(params["dp0"]["b"]), wdp8, bd8)

    return dec[:, :3]


# speedup vs baseline: 5.6707x; 1.4373x over previous
import functools

import jax
import jax.numpy as jnp
from jax import lax
from jax.experimental import pallas as pl
from jax.experimental.pallas import tpu as pltpu
from jax.experimental.pallas import tpu_sc as plsc

N = 10000
E = 320000
H = 128

NC = 2
NS = 16
NW = NC * NS
SC_B = 80
PER_W = E // NW
SC_ITERS = PER_W // SC_B
ROW_A = 624
ROW_B = N - 15 * ROW_A

_mesh = plsc.VectorSubcoreMesh(core_axis_name="c", subcore_axis_name="s")


G_B = 128
G_CHUNKS = 79
G_LAST = PER_W - G_B


def _gather2(t1, t2, sidx, ridx):
    D = t1.shape[1]
    out = jax.ShapeDtypeStruct((E, D), jnp.float32)
    slot_scratch = [
        pltpu.VMEM((G_B,), jnp.int32),
        pltpu.VMEM((G_B,), jnp.int32),
        pltpu.VMEM((G_B, D), jnp.float32),
        pltpu.VMEM((G_B, D), jnp.float32),
        pltpu.SemaphoreType.DMA,
        pltpu.SemaphoreType.DMA,
    ]

    @functools.partial(
        pl.kernel,
        out_type=(out, out),
        mesh=_mesh,
        scratch_types=slot_scratch + slot_scratch,
    )
    def k(t1_h, t2_h, s_h, r_h, o1_h, o2_h,
          si0, ri0, r10, r20, sa0, sb0, si1, ri1, r11, r21, sa1, sb1):
        wid = lax.axis_index("s") * NC + lax.axis_index("c")
        w0 = wid * PER_W
        si = (si0, si1)
        ri = (ri0, ri1)
        r1 = (r10, r11)
        r2 = (r20, r21)
        sa = (sa0, sa1)
        sb = (sb0, sb1)

        def off(j):
            return w0 + jnp.where(j < G_CHUNKS - 1, j * G_B, G_LAST)

        def start(j, slot):
            base = off(j)
            pltpu.sync_copy(s_h.at[pl.ds(base, G_B)], si[slot])
            pltpu.sync_copy(r_h.at[pl.ds(base, G_B)], ri[slot])
            pltpu.async_copy(t1_h.at[si[slot]], r1[slot], sa[slot])
            pltpu.async_copy(t2_h.at[ri[slot]], r2[slot], sb[slot])

        def finish(j, slot):
            base = off(j)
            pltpu.make_async_copy(t1_h.at[si[slot]], r1[slot], sa[slot]).wait()
            pltpu.make_async_copy(t2_h.at[ri[slot]], r2[slot], sb[slot]).wait()
            pltpu.sync_copy(r1[slot], o1_h.at[pl.ds(base, G_B)])
            pltpu.sync_copy(r2[slot], o2_h.at[pl.ds(base, G_B)])

        start(0, 0)

        def body(t, carry):
            j0 = 2 * t
            start(j0 + 1, 1)
            finish(j0, 0)
            start(j0 + 2, 0)
            finish(j0 + 1, 1)
            return carry

        lax.fori_loop(0, (G_CHUNKS - 1) // 2, body, 0)
        finish(G_CHUNKS - 1, 0)

    return k(t1, t2, sidx, ridx)


HN = N // NC
TRASH = HN
ACC_R = HN + 8
PER_S = E // NS
S_ITERS = PER_S // SC_B
WB_A = 312
WB_B = HN - 15 * WB_A


def _scatter_add(vals, ridx):
    @functools.partial(
        pl.kernel,
        out_type=jax.ShapeDtypeStruct((N, H), jnp.float32),
        mesh=_mesh,
        scratch_types=[
            pltpu.VMEM((SC_B,), jnp.int32),
            pltpu.VMEM((SC_B, H), jnp.float32),
            pltpu.VMEM((SC_B,), jnp.int32),
            pltpu.VMEM((SC_B, H), jnp.float32),
            pltpu.VMEM((WB_B, H), jnp.float32),
            pltpu.VMEM_SHARED((ACC_R, H), jnp.float32),
            pltpu.SemaphoreType.DMA,
            pltpu.SemaphoreType.DMA,
        ],
    )
    def k(v_h, r_h, o_h, idx0, rows0, idx1, rows1, zbuf, acc, sm0, sm1):
        c = lax.axis_index("c")
        s = lax.axis_index("s")
        lo = c * HN

        def zrow(i, carry):
            def zcol(j, cc):
                zbuf[i, pl.ds(j * 16, 16)] = jnp.zeros((16,), jnp.float32)
                return cc
            return lax.fori_loop(0, H // 16, zcol, carry)

        lax.fori_loop(0, WB_B, zrow, 0)

        @pl.when(s < 15)
        def _():
            pltpu.sync_copy(zbuf.at[pl.ds(0, WB_A)],
                            acc.at[pl.ds(s * WB_A, WB_A)])

        @pl.when(s == 15)
        def _():
            pltpu.sync_copy(zbuf, acc.at[pl.ds(15 * WB_A, WB_B)])

        plsc.subcore_barrier()

        idx = (idx0, idx1)
        rows = (rows0, rows1)
        sm = (sm0, sm1)

        def start(j, slot):
            base = s * PER_S + j * SC_B
            pltpu.async_copy(v_h.at[pl.ds(base, SC_B)], rows[slot], sm[slot])
            pltpu.sync_copy(r_h.at[pl.ds(base, SC_B)], idx[slot])

        def finish(j, slot):
            base = s * PER_S + j * SC_B
            pltpu.make_async_copy(v_h.at[pl.ds(base, SC_B)], rows[slot],
                                  sm[slot]).wait()
            for t in range(SC_B // 16):
                v = idx[slot][pl.ds(t * 16, 16)] - lo
                ok = (v >= 0) & (v < HN)
                idx[slot][pl.ds(t * 16, 16)] = jnp.where(ok, v, TRASH)
            pltpu.sync_copy(rows[slot], acc.at[idx[slot]], add=True)

        start(0, 0)

        def body(t, carry):
            j0 = 2 * t
            start(j0 + 1, 1)
            finish(j0, 0)
            start(j0 + 2, 0)
            finish(j0 + 1, 1)
            return carry

        lax.fori_loop(0, (S_ITERS - 2) // 2, body, 0)
        start(S_ITERS - 1, 1)
        finish(S_ITERS - 2, 0)
        finish(S_ITERS - 1, 1)
        plsc.subcore_barrier()

        @pl.when(s < 15)
        def _():
            pltpu.sync_copy(acc.at[pl.ds(s * WB_A, WB_A)],
                            o_h.at[pl.ds(lo + s * WB_A, WB_A)])

        @pl.when(s == 15)
        def _():
            pltpu.sync_copy(acc.at[pl.ds(15 * WB_A, WB_B)],
                            o_h.at[pl.ds(lo + 15 * WB_A, WB_B)])

    return k(vals, ridx)


def _ln(h, g, b):
    m = jnp.mean(h, axis=-1, keepdims=True)
    v = jnp.mean((h - m) * (h - m), axis=-1, keepdims=True)
    return (h - m) * lax.rsqrt(v + 1e-5) * g + b


def _dot(a, b):
    return jnp.dot(a, b, preferred_element_type=jnp.float32)


def _full(shape):
    return pl.BlockSpec(shape, lambda i: (0,) * len(shape))


def _rows(blk, d):
    return pl.BlockSpec((blk, d), lambda i: (i, 0))


N_BLK = 2000
E_BLK = 2560


def _tc_enc_node(x16, w0, b0, w1, b1, g, bl, wa, wb):
    def body(x_r, w0_r, b0_r, w1_r, b1_r, g_r, bl_r, wa_r, wb_r,
             xh_r, a1_r, a2_r):
        h = jnp.maximum(_dot(x_r[...], w0_r[...]) + b0_r[...], 0.0)
        xh = _ln(_dot(h, w1_r[...]) + b1_r[...], g_r[...], bl_r[...])
        xh_r[...] = xh
        a1_r[...] = _dot(xh, wa_r[...])
        a2_r[...] = _dot(xh, wb_r[...])

    o = jax.ShapeDtypeStruct((N, H), jnp.float32)
    return pl.pallas_call(
        body,
        grid=(N // N_BLK,),
        in_specs=[_rows(N_BLK, 16), _full((16, H)), _full((1, H)),
                  _full((H, H)), _full((1, H)), _full((1, H)), _full((1, H)),
                  _full((H, H)), _full((H, H))],
        out_specs=[_rows(N_BLK, H)] * 3,
        out_shape=[o, o, o],
    )(x16, w0, b0, w1, b1, g, bl, wa, wb)


def _tc_enc_edge(gs, gr, wlin, wd, wdw, b0, w1, b1, g, bl):
    def body(gs_r, gr_r, wlin_r, wd_r, wdw_r, b0_r, w1_r, b1_r, g_r, bl_r,
             eh_r):
        rel = gs_r[...] - gr_r[...]
        d2 = rel[:, 0:1] * rel[:, 0:1] + rel[:, 1:2] * rel[:, 1:2]
        dw2 = rel[:, 2:3] * rel[:, 2:3] + rel[:, 3:4] * rel[:, 3:4]
        pre = (_dot(rel, wlin_r[...])
               + jnp.sqrt(d2) * wd_r[...]
               + jnp.sqrt(dw2) * wdw_r[...] + b0_r[...])
        h = jnp.maximum(pre, 0.0)
        eh_r[...] = _ln(_dot(h, w1_r[...]) + b1_r[...], g_r[...], bl_r[...])

    return pl.pallas_call(
        body,
        grid=(E // E_BLK,),
        in_specs=[_rows(E_BLK, H), _rows(E_BLK, H), _full((H, H)),
                  _full((1, H)), _full((1, H)), _full((1, H)),
                  _full((H, H)), _full((1, H)), _full((1, H)), _full((1, H))],
        out_specs=[_rows(E_BLK, H)],
        out_shape=[jax.ShapeDtypeStruct((E, H), jnp.float32)],
    )(gs, gr, wlin, wd, wdw, b0, w1, b1, g, bl)[0]


def _tc_edge_step(g1, g2, eh, w3, b0, w1, b1, g, bl):
    def body(g1_r, g2_r, eh_r, w3_r, b0_r, w1_r, b1_r, g_r, bl_r,
             ne_r, en_r):
        eh_v = eh_r[...]
        t = jnp.maximum(g1_r[...] + g2_r[...] + _dot(eh_v, w3_r[...])
                        + b0_r[...], 0.0)
        t = jnp.maximum(_dot(t, w1_r[...]) + b1_r[...], 0.0)
        ne = _ln(t, g_r[...], bl_r[...])
        ne_r[...] = ne
        en_r[...] = ne + eh_v

    o = jax.ShapeDtypeStruct((E, H), jnp.float32)
    return pl.pallas_call(
        body,
        grid=(E // E_BLK,),
        in_specs=[_rows(E_BLK, H)] * 3 + [_full((H, H)), _full((1, H)),
                  _full((H, H)), _full((1, H)), _full((1, H)), _full((1, H))],
        out_specs=[_rows(E_BLK, H)] * 2,
        out_shape=[o, o],
    )(g1, g2, eh, w3, b0, w1, b1, g, bl)


def _tc_node_step(xh, aggr, w0a, w0b, b0, w1, b1, g, bl, wa, wb):
    def body(xh_r, ag_r, w0a_r, w0b_r, b0_r, w1_r, b1_r, g_r, bl_r,
             wa_r, wb_r, xn_r, a1_r, a2_r):
        xh_v = xh_r[...]
        t = jnp.maximum(_dot(xh_v, w0a_r[...]) + _dot(ag_r[...], w0b_r[...])
                        + b0_r[...], 0.0)
        t = jnp.maximum(_dot(t, w1_r[...]) + b1_r[...], 0.0)
        xn = _ln(t, g_r[...], bl_r[...]) + xh_v
        xn_r[...] = xn
        a1_r[...] = _dot(xn, wa_r[...])
        a2_r[...] = _dot(xn, wb_r[...])

    o = jax.ShapeDtypeStruct((N, H), jnp.float32)
    return pl.pallas_call(
        body,
        grid=(N // N_BLK,),
        in_specs=[_rows(N_BLK, H)] * 2 + [_full((H, H)), _full((H, H)),
                  _full((1, H)), _full((H, H)), _full((1, H)), _full((1, H)),
                  _full((1, H)), _full((H, H)), _full((H, H))],
        out_specs=[_rows(N_BLK, H)] * 3,
        out_shape=[o, o, o],
    )(xh, aggr, w0a, w0b, b0, w1, b1, g, bl, wa, wb)


def _tc_node_last(xh, aggr, w0a, w0b, b0, w1, b1, g, bl,
                  dw0, dbw0, wdw8, dp0, dbp0, wdp8, bd8):
    def body(xh_r, ag_r, w0a_r, w0b_r, b0_r, w1_r, b1_r, g_r, bl_r,
             dw0_r, dbw0_r, wdw8_r, dp0_r, dbp0_r, wdp8_r, bd8_r, o_r):
        xh_v = xh_r[...]
        t = jnp.maximum(_dot(xh_v, w0a_r[...]) + _dot(ag_r[...], w0b_r[...])
                        + b0_r[...], 0.0)
        t = jnp.maximum(_dot(t, w1_r[...]) + b1_r[...], 0.0)
        xn = _ln(t, g_r[...], bl_r[...]) + xh_v
        d1 = jnp.maximum(_dot(xn, dw0_r[...]) + dbw0_r[...], 0.0)
        d2 = jnp.maximum(_dot(xn, dp0_r[...]) + dbp0_r[...], 0.0)
        o_r[...] = _dot(d1, wdw8_r[...]) + _dot(d2, wdp8_r[...]) + bd8_r[...]

    return pl.pallas_call(
        body,
        grid=(N // N_BLK,),
        in_specs=[_rows(N_BLK, H)] * 2 + [_full((H, H)), _full((H, H)),
                  _full((1, H)), _full((H, H)), _full((1, H)), _full((1, H)),
                  _full((1, H)), _full((H, H)), _full((1, H)), _full((H, 8)),
                  _full((H, H)), _full((1, H)), _full((H, 8)), _full((1, 8))],
        out_specs=[_rows(N_BLK, 8)],
        out_shape=[jax.ShapeDtypeStruct((N, 8), jnp.float32)],
    )(xh, aggr, w0a, w0b, b0, w1, b1, g, bl,
      dw0, dbw0, wdw8, dp0, dbp0, wdp8, bd8)[0]


def kernel(world_pos, mesh_pos, prev_world_pos, phi, prev_phi, swelling_phi,
           swelling_phi_rate, swelling_phi_rate_prev, node_type, mat_param,
           edge_index, params):
    f32 = jnp.float32
    senders = edge_index[0].astype(jnp.int32)
    receivers = edge_index[1].astype(jnp.int32)

    x16 = jnp.concatenate(
        [phi, prev_phi, swelling_phi, swelling_phi_rate,
         swelling_phi_rate_prev, node_type,
         jnp.zeros((N, 2), f32)], axis=1)
    ne0w = params["ne0"]["w"]
    w0p = jnp.concatenate(
        [(ne0w[0] + ne0w[1])[None], (-ne0w[1])[None], ne0w[2:],
         jnp.zeros((2, H), f32)], axis=0)

    P = jnp.concatenate([mesh_pos, world_pos, phi, jnp.zeros((N, H - 5), f32)],
                        axis=1)
    ee0w = params["ee0"]["w"]
    wlin = jnp.concatenate([ee0w[0:2], ee0w[3:5], ee0w[6:7],
                            jnp.zeros((H - 5, H), f32)], axis=0)
    wd = ee0w[2][None]
    wdw = ee0w[5][None]

    def r1(v):
        return v[None]

    pr0 = params["procs"][0]
    x_h, a1, a2 = _tc_enc_node(
        x16, w0p, r1(params["ne0"]["b"]), params["ne1"]["w"],
        r1(params["ne1"]["b"]), r1(params["ne_ln"]["g"]),
        r1(params["ne_ln"]["b"]),
        pr0["e0"]["w"][0:H], pr0["e0"]["w"][H:2 * H])

    gs, gr = _gather2(P, P, senders, receivers)
    e_h = _tc_enc_edge(
        gs, gr, wlin, wd, wdw, r1(params["ee0"]["b"]), params["ee1"]["w"],
        r1(params["ee1"]["b"]), r1(params["ee_ln"]["g"]),
        r1(params["ee_ln"]["b"]))

    dec = None
    for k in range(3):
        pr = params["procs"][k]
        g1, g2 = _gather2(a1, a2, senders, receivers)
        new_e, e_h = _tc_edge_step(
            g1, g2, e_h, pr["e0"]["w"][2 * H:3 * H], r1(pr["e0"]["b"]),
            pr["e1"]["w"], r1(pr["e1"]["b"]), r1(pr["e_ln"]["g"]),
            r1(pr["e_ln"]["b"]))
        aggr = _scatter_add(new_e, receivers)
        nargs = (x_h, aggr, pr["n0"]["w"][0:H], pr["n0"]["w"][H:2 * H],
                 r1(pr["n0"]["b"]), pr["n1"]["w"], r1(pr["n1"]["b"]),
                 r1(pr["n_ln"]["g"]), r1(pr["n_ln"]["b"]))
        if k < 2:
            prn = params["procs"][k + 1]
            x_h, a1, a2 = _tc_node_step(
                *nargs, prn["e0"]["w"][0:H], prn["e0"]["w"][H:2 * H])
        else:
            wdw8 = jnp.zeros((H, 8), f32).at[:, 0:2].set(params["dw1"]["w"])
            wdp8 = jnp.zeros((H, 8), f32).at[:, 2:3].set(params["dp1"]["w"])
            bd8 = jnp.zeros((1, 8), f32).at[0, 0:2].set(
                params["dw1"]["b"]).at[0, 2].set(params["dp1"]["b"][0])
            dec = _tc_node_last(
                *nargs, params["dw0"]["w"], r1(params["dw0"]["b"]), wdw8,
                params["dp0"]["w"], r1(params["dp0"]["b"]), wdp8, bd8)

    return dec[:, :3]


# speedup vs baseline: 6.3474x; 1.1193x over previous
import functools

import jax
import jax.numpy as jnp
from jax import lax
from jax.experimental import pallas as pl
from jax.experimental.pallas import tpu as pltpu
from jax.experimental.pallas import tpu_sc as plsc

N = 10000
E = 320000
H = 128

NC = 2
NS = 16
NW = NC * NS
SC_B = 80
PER_W = E // NW
SC_ITERS = PER_W // SC_B
ROW_A = 624
ROW_B = N - 15 * ROW_A

_mesh = plsc.VectorSubcoreMesh(core_axis_name="c", subcore_axis_name="s")


G_B = 128


def _pipe(C, start, finish):
    start(0, 0)

    def body(t, carry):
        j0 = 2 * t
        start(j0 + 1, 1)
        finish(j0, 0)
        start(j0 + 2, 0)
        finish(j0 + 1, 1)
        return carry

    if C % 2 == 1:
        lax.fori_loop(0, (C - 1) // 2, body, 0)
        finish(C - 1, 0)
    else:
        lax.fori_loop(0, (C - 2) // 2, body, 0)
        start(C - 1, 1)
        finish(C - 2, 0)
        finish(C - 1, 1)


def _gather2(t1, t2, sidx, ridx):
    D = t1.shape[1]
    EQ = sidx.shape[0]
    span = EQ // NW
    n_chunks = -(-span // G_B)
    last_off = span - G_B
    out = jax.ShapeDtypeStruct((EQ, D), jnp.float32)
    slot_scratch = [
        pltpu.VMEM((G_B,), jnp.int32),
        pltpu.VMEM((G_B,), jnp.int32),
        pltpu.VMEM((G_B, D), jnp.float32),
        pltpu.VMEM((G_B, D), jnp.float32),
        pltpu.SemaphoreType.DMA,
        pltpu.SemaphoreType.DMA,
    ]

    @functools.partial(
        pl.kernel,
        out_type=(out, out),
        mesh=_mesh,
        scratch_types=slot_scratch + slot_scratch,
    )
    def k(t1_h, t2_h, s_h, r_h, o1_h, o2_h,
          si0, ri0, r10, r20, sa0, sb0, si1, ri1, r11, r21, sa1, sb1):
        wid = lax.axis_index("s") * NC + lax.axis_index("c")
        w0 = wid * span
        si = (si0, si1)
        ri = (ri0, ri1)
        r1 = (r10, r11)
        r2 = (r20, r21)
        sa = (sa0, sa1)
        sb = (sb0, sb1)

        def off(j):
            return w0 + jnp.where(j < n_chunks - 1, j * G_B, last_off)

        def start(j, slot):
            base = off(j)
            pltpu.sync_copy(s_h.at[pl.ds(base, G_B)], si[slot])
            pltpu.sync_copy(r_h.at[pl.ds(base, G_B)], ri[slot])
            pltpu.async_copy(t1_h.at[si[slot]], r1[slot], sa[slot])
            pltpu.async_copy(t2_h.at[ri[slot]], r2[slot], sb[slot])

        def finish(j, slot):
            base = off(j)
            pltpu.make_async_copy(t1_h.at[si[slot]], r1[slot], sa[slot]).wait()
            pltpu.make_async_copy(t2_h.at[ri[slot]], r2[slot], sb[slot]).wait()
            pltpu.sync_copy(r1[slot], o1_h.at[pl.ds(base, G_B)])
            pltpu.sync_copy(r2[slot], o2_h.at[pl.ds(base, G_B)])

        _pipe(n_chunks, start, finish)

    return k(t1, t2, sidx, ridx)


HN = N // NC
TRASH = HN
ACC_R = HN + 8
WB_A = 312
WB_B = HN - 15 * WB_A


S_B = 128


def _scatter_add(vals, ridx):
    EQ = ridx.shape[0]
    per_s = EQ // NS
    s_full = per_s // S_B
    s_tail = per_s - s_full * S_B

    @functools.partial(
        pl.kernel,
        out_type=jax.ShapeDtypeStruct((N, H), jnp.float32),
        mesh=_mesh,
        scratch_types=[
            pltpu.VMEM((S_B,), jnp.int32),
            pltpu.VMEM((S_B, H), jnp.float32),
            pltpu.VMEM((S_B,), jnp.int32),
            pltpu.VMEM((S_B, H), jnp.float32),
            pltpu.VMEM((16,), jnp.int32),
            pltpu.VMEM((16, H), jnp.float32),
            pltpu.VMEM((WB_B, H), jnp.float32),
            pltpu.VMEM_SHARED((ACC_R, H), jnp.float32),
            pltpu.SemaphoreType.DMA,
            pltpu.SemaphoreType.DMA,
        ],
    )
    def k(v_h, r_h, o_h, idx0, rows0, idx1, rows1, idxt, rowst, zbuf, acc,
          sm0, sm1):
        c = lax.axis_index("c")
        s = lax.axis_index("s")
        lo = c * HN

        def zrow(i, carry):
            def zcol(j, cc):
                zbuf[i, pl.ds(j * 16, 16)] = jnp.zeros((16,), jnp.float32)
                return cc
            return lax.fori_loop(0, H // 16, zcol, carry)

        lax.fori_loop(0, WB_B, zrow, 0)

        @pl.when(s < 15)
        def _():
            pltpu.sync_copy(zbuf.at[pl.ds(0, WB_A)],
                            acc.at[pl.ds(s * WB_A, WB_A)])

        @pl.when(s == 15)
        def _():
            pltpu.sync_copy(zbuf, acc.at[pl.ds(15 * WB_A, WB_B)])

        plsc.subcore_barrier()

        idx = (idx0, idx1)
        rows = (rows0, rows1)
        sm = (sm0, sm1)

        def rebase(ref, nvec):
            for t in range(nvec):
                v = ref[pl.ds(t * 16, 16)] - lo
                ok = (v >= 0) & (v < HN)
                ref[pl.ds(t * 16, 16)] = jnp.where(ok, v, TRASH)

        def start(j, slot):
            base = s * per_s + j * S_B
            pltpu.async_copy(v_h.at[pl.ds(base, S_B)], rows[slot], sm[slot])
            pltpu.sync_copy(r_h.at[pl.ds(base, S_B)], idx[slot])

        def finish(j, slot):
            base = s * per_s + j * S_B
            pltpu.make_async_copy(v_h.at[pl.ds(base, S_B)], rows[slot],
                                  sm[slot]).wait()
            rebase(idx[slot], S_B // 16)
            pltpu.sync_copy(rows[slot], acc.at[idx[slot]], add=True)

        _pipe(s_full, start, finish)
        if s_tail:
            tbase = s * per_s + s_full * S_B
            pltpu.sync_copy(v_h.at[pl.ds(tbase, s_tail)], rowst)
            pltpu.sync_copy(r_h.at[pl.ds(tbase, s_tail)], idxt)
            rebase(idxt, s_tail // 16)
            pltpu.sync_copy(rowst, acc.at[idxt], add=True)
        plsc.subcore_barrier()

        @pl.when(s < 15)
        def _():
            pltpu.sync_copy(acc.at[pl.ds(s * WB_A, WB_A)],
                            o_h.at[pl.ds(lo + s * WB_A, WB_A)])

        @pl.when(s == 15)
        def _():
            pltpu.sync_copy(acc.at[pl.ds(15 * WB_A, WB_B)],
                            o_h.at[pl.ds(lo + 15 * WB_A, WB_B)])

    return k(vals, ridx)


def _ln(h, g, b):
    m = jnp.mean(h, axis=-1, keepdims=True)
    v = jnp.mean((h - m) * (h - m), axis=-1, keepdims=True)
    return (h - m) * lax.rsqrt(v + 1e-5) * g + b


def _dot(a, b):
    return jnp.dot(a, b, preferred_element_type=jnp.float32)


def _full(shape):
    return pl.BlockSpec(shape, lambda i: (0,) * len(shape))


def _rows(blk, d):
    return pl.BlockSpec((blk, d), lambda i: (i, 0))


N_BLK = 2000
E_BLK = 2000


def _tc_enc_node(x16, w0, b0, w1, b1, g, bl, wa, wb):
    def body(x_r, w0_r, b0_r, w1_r, b1_r, g_r, bl_r, wa_r, wb_r,
             xh_r, a1_r, a2_r):
        h = jnp.maximum(_dot(x_r[...], w0_r[...]) + b0_r[...], 0.0)
        xh = _ln(_dot(h, w1_r[...]) + b1_r[...], g_r[...], bl_r[...])
        xh_r[...] = xh
        a1_r[...] = _dot(xh, wa_r[...])
        a2_r[...] = _dot(xh, wb_r[...])

    o = jax.ShapeDtypeStruct((N, H), jnp.float32)
    return pl.pallas_call(
        body,
        grid=(N // N_BLK,),
        in_specs=[_rows(N_BLK, 16), _full((16, H)), _full((1, H)),
                  _full((H, H)), _full((1, H)), _full((1, H)), _full((1, H)),
                  _full((H, H)), _full((H, H))],
        out_specs=[_rows(N_BLK, H)] * 3,
        out_shape=[o, o, o],
    )(x16, w0, b0, w1, b1, g, bl, wa, wb)


def _tc_enc_edge(gs, gr, wlin, wd, wdw, b0, w1, b1, g, bl):
    def body(gs_r, gr_r, wlin_r, wd_r, wdw_r, b0_r, w1_r, b1_r, g_r, bl_r,
             eh_r):
        rel = gs_r[...] - gr_r[...]
        d2 = rel[:, 0:1] * rel[:, 0:1] + rel[:, 1:2] * rel[:, 1:2]
        dw2 = rel[:, 2:3] * rel[:, 2:3] + rel[:, 3:4] * rel[:, 3:4]
        pre = (_dot(rel, wlin_r[...])
               + jnp.sqrt(d2) * wd_r[...]
               + jnp.sqrt(dw2) * wdw_r[...] + b0_r[...])
        h = jnp.maximum(pre, 0.0)
        eh_r[...] = _ln(_dot(h, w1_r[...]) + b1_r[...], g_r[...], bl_r[...])

    EQ = gs.shape[0]
    return pl.pallas_call(
        body,
        grid=(EQ // E_BLK,),
        in_specs=[_rows(E_BLK, H), _rows(E_BLK, H), _full((H, H)),
                  _full((1, H)), _full((1, H)), _full((1, H)),
                  _full((H, H)), _full((1, H)), _full((1, H)), _full((1, H))],
        out_specs=[_rows(E_BLK, H)],
        out_shape=[jax.ShapeDtypeStruct((EQ, H), jnp.float32)],
    )(gs, gr, wlin, wd, wdw, b0, w1, b1, g, bl)[0]


def _tc_edge_step(g1, g2, eh, w3, b0, w1, b1, g, bl):
    def body(g1_r, g2_r, eh_r, w3_r, b0_r, w1_r, b1_r, g_r, bl_r,
             ne_r, en_r):
        eh_v = eh_r[...]
        t = jnp.maximum(g1_r[...] + g2_r[...] + _dot(eh_v, w3_r[...])
                        + b0_r[...], 0.0)
        t = jnp.maximum(_dot(t, w1_r[...]) + b1_r[...], 0.0)
        ne = _ln(t, g_r[...], bl_r[...])
        ne_r[...] = ne
        en_r[...] = ne + eh_v

    EQ = g1.shape[0]
    o = jax.ShapeDtypeStruct((EQ, H), jnp.float32)
    return pl.pallas_call(
        body,
        grid=(EQ // E_BLK,),
        in_specs=[_rows(E_BLK, H)] * 3 + [_full((H, H)), _full((1, H)),
                  _full((H, H)), _full((1, H)), _full((1, H)), _full((1, H))],
        out_specs=[_rows(E_BLK, H)] * 2,
        out_shape=[o, o],
    )(g1, g2, eh, w3, b0, w1, b1, g, bl)


def _tc_node_step(xh, p0, p1, w0a, w0b, b0, w1, b1, g, bl, wa, wb):
    def body(xh_r, p0_r, p1_r, w0a_r, w0b_r, b0_r, w1_r, b1_r, g_r, bl_r,
             wa_r, wb_r, xn_r, a1_r, a2_r):
        xh_v = xh_r[...]
        ag = p0_r[...] + p1_r[...]
        t = jnp.maximum(_dot(xh_v, w0a_r[...]) + _dot(ag, w0b_r[...])
                        + b0_r[...], 0.0)
        t = jnp.maximum(_dot(t, w1_r[...]) + b1_r[...], 0.0)
        xn = _ln(t, g_r[...], bl_r[...]) + xh_v
        xn_r[...] = xn
        a1_r[...] = _dot(xn, wa_r[...])
        a2_r[...] = _dot(xn, wb_r[...])

    o = jax.ShapeDtypeStruct((N, H), jnp.float32)
    return pl.pallas_call(
        body,
        grid=(N // N_BLK,),
        in_specs=[_rows(N_BLK, H)] * 3 + [_full((H, H)), _full((H, H)),
                  _full((1, H)), _full((H, H)), _full((1, H)), _full((1, H)),
                  _full((1, H)), _full((H, H)), _full((H, H))],
        out_specs=[_rows(N_BLK, H)] * 3,
        out_shape=[o, o, o],
    )(xh, p0, p1, w0a, w0b, b0, w1, b1, g, bl, wa, wb)


def _tc_node_last(xh, p0, p1, w0a, w0b, b0, w1, b1, g, bl,
                  dw0, dbw0, wdw8, dp0, dbp0, wdp8, bd8):
    def body(xh_r, p0_r, p1_r, w0a_r, w0b_r, b0_r, w1_r, b1_r, g_r, bl_r,
             dw0_r, dbw0_r, wdw8_r, dp0_r, dbp0_r, wdp8_r, bd8_r, o_r):
        xh_v = xh_r[...]
        ag = p0_r[...] + p1_r[...]
        t = jnp.maximum(_dot(xh_v, w0a_r[...]) + _dot(ag, w0b_r[...])
                        + b0_r[...], 0.0)
        t = jnp.maximum(_dot(t, w1_r[...]) + b1_r[...], 0.0)
        xn = _ln(t, g_r[...], bl_r[...]) + xh_v
        d1 = jnp.maximum(_dot(xn, dw0_r[...]) + dbw0_r[...], 0.0)
        d2 = jnp.maximum(_dot(xn, dp0_r[...]) + dbp0_r[...], 0.0)
        o_r[...] = _dot(d1, wdw8_r[...]) + _dot(d2, wdp8_r[...]) + bd8_r[...]

    return pl.pallas_call(
        body,
        grid=(N // N_BLK,),
        in_specs=[_rows(N_BLK, H)] * 3 + [_full((H, H)), _full((H, H)),
                  _full((1, H)), _full((H, H)), _full((1, H)), _full((1, H)),
                  _full((1, H)), _full((H, H)), _full((1, H)), _full((H, 8)),
                  _full((H, H)), _full((1, H)), _full((H, 8)), _full((1, 8))],
        out_specs=[_rows(N_BLK, 8)],
        out_shape=[jax.ShapeDtypeStruct((N, 8), jnp.float32)],
    )(xh, p0, p1, w0a, w0b, b0, w1, b1, g, bl,
      dw0, dbw0, wdw8, dp0, dbp0, wdp8, bd8)[0]


def kernel(world_pos, mesh_pos, prev_world_pos, phi, prev_phi, swelling_phi,
           swelling_phi_rate, swelling_phi_rate_prev, node_type, mat_param,
           edge_index, params):
    f32 = jnp.float32
    senders = edge_index[0].astype(jnp.int32)
    receivers = edge_index[1].astype(jnp.int32)

    x16 = jnp.concatenate(
        [phi, prev_phi, swelling_phi, swelling_phi_rate,
         swelling_phi_rate_prev, node_type,
         jnp.zeros((N, 2), f32)], axis=1)
    ne0w = params["ne0"]["w"]
    w0p = jnp.concatenate(
        [(ne0w[0] + ne0w[1])[None], (-ne0w[1])[None], ne0w[2:],
         jnp.zeros((2, H), f32)], axis=0)

    P = jnp.concatenate([mesh_pos, world_pos, phi, jnp.zeros((N, H - 5), f32)],
                        axis=1)
    ee0w = params["ee0"]["w"]
    wlin = jnp.concatenate([ee0w[0:2], ee0w[3:5], ee0w[6:7],
                            jnp.zeros((H - 5, H), f32)], axis=0)
    wd = ee0w[2][None]
    wdw = ee0w[5][None]

    def r1(v):
        return v[None]

    pr0 = params["procs"][0]
    x_h, a1, a2 = _tc_enc_node(
        x16, w0p, r1(params["ne0"]["b"]), params["ne1"]["w"],
        r1(params["ne1"]["b"]), r1(params["ne_ln"]["g"]),
        r1(params["ne_ln"]["b"]),
        pr0["e0"]["w"][0:H], pr0["e0"]["w"][H:2 * H])

    E2 = E // 2
    sid = (senders[:E2], senders[E2:])
    rid = (receivers[:E2], receivers[E2:])

    e_h = []
    for q in range(2):
        gs, gr = _gather2(P, P, sid[q], rid[q])
        e_h.append(_tc_enc_edge(
            gs, gr, wlin, wd, wdw, r1(params["ee0"]["b"]),
            params["ee1"]["w"], r1(params["ee1"]["b"]),
            r1(params["ee_ln"]["g"]), r1(params["ee_ln"]["b"])))

    dec = None
    for k in range(3):
        pr = params["procs"][k]
        part = []
        for q in range(2):
            g1, g2 = _gather2(a1, a2, sid[q], rid[q])
            new_e, e_h[q] = _tc_edge_step(
                g1, g2, e_h[q], pr["e0"]["w"][2 * H:3 * H], r1(pr["e0"]["b"]),
                pr["e1"]["w"], r1(pr["e1"]["b"]), r1(pr["e_ln"]["g"]),
                r1(pr["e_ln"]["b"]))
            part.append(_scatter_add(new_e, rid[q]))
        nargs = (x_h, part[0], part[1], pr["n0"]["w"][0:H],
                 pr["n0"]["w"][H:2 * H],
                 r1(pr["n0"]["b"]), pr["n1"]["w"], r1(pr["n1"]["b"]),
                 r1(pr["n_ln"]["g"]), r1(pr["n_ln"]["b"]))
        if k < 2:
            prn = params["procs"][k + 1]
            x_h, a1, a2 = _tc_node_step(
                *nargs, prn["e0"]["w"][0:H], prn["e0"]["w"][H:2 * H])
        else:
            wdw8 = jnp.zeros((H, 8), f32).at[:, 0:2].set(params["dw1"]["w"])
            wdp8 = jnp.zeros((H, 8), f32).at[:, 2:3].set(params["dp1"]["w"])
            bd8 = jnp.zeros((1, 8), f32).at[0, 0:2].set(
                params["dw1"]["b"]).at[0, 2].set(params["dp1"]["b"][0])
            dec = _tc_node_last(
                *nargs, params["dw0"]["w"], r1(params["dw0"]["b"]), wdw8,
                params["dp0"]["w"], r1(params["dp0"]["b"]), wdp8, bd8)

    return dec[:, :3]


# speedup vs baseline: 7.5679x; 1.1923x over previous
import functools

import jax
import jax.numpy as jnp
from jax import lax
from jax.experimental import pallas as pl
from jax.experimental.pallas import tpu as pltpu
from jax.experimental.pallas import tpu_sc as plsc

N = 10000
E = 320000
H = 128

NC = 2
NS = 16
NW = NC * NS
SC_B = 80
PER_W = E // NW
SC_ITERS = PER_W // SC_B
ROW_A = 624
ROW_B = N - 15 * ROW_A

_mesh = plsc.VectorSubcoreMesh(core_axis_name="c", subcore_axis_name="s")


G_B = 128


def _pipe(C, start, finish):
    start(0, 0)

    def body(t, carry):
        j0 = 2 * t
        start(j0 + 1, 1)
        finish(j0, 0)
        start(j0 + 2, 0)
        finish(j0 + 1, 1)
        return carry

    if C % 2 == 1:
        lax.fori_loop(0, (C - 1) // 2, body, 0)
        finish(C - 1, 0)
    else:
        lax.fori_loop(0, (C - 2) // 2, body, 0)
        start(C - 1, 1)
        finish(C - 2, 0)
        finish(C - 1, 1)


def _gather_sum(t1, t2, sidx, ridx):
    D = t1.shape[1]
    EQ = sidx.shape[0]
    span = EQ // NW
    n_chunks = -(-span // G_B)
    last_off = span - G_B
    out = jax.ShapeDtypeStruct((EQ, D), jnp.float32)
    slot_scratch = [
        pltpu.VMEM((G_B,), jnp.int32),
        pltpu.VMEM((G_B,), jnp.int32),
        pltpu.VMEM((G_B, D), jnp.float32),
        pltpu.SemaphoreType.DMA,
        pltpu.SemaphoreType.DMA,
    ]

    @functools.partial(
        pl.kernel,
        out_type=out,
        mesh=_mesh,
        scratch_types=slot_scratch * 3,
    )
    def k(t1_h, t2_h, s_h, r_h, o_h,
          si0, ri0, r0, sa0, sb0, si1, ri1, rr1, sa1, sb1,
          si2, ri2, r2, sa2, sb2):
        wid = lax.axis_index("s") * NC + lax.axis_index("c")
        w0 = wid * span
        si = (si0, si1, si2)
        ri = (ri0, ri1, ri2)
        rb = (r0, rr1, r2)
        sa = (sa0, sa1, sa2)
        sb = (sb0, sb1, sb2)
        C = n_chunks

        def off(j):
            return w0 + jnp.where(j < C - 1, j * G_B, last_off)

        def start(j, slot):
            j = jnp.int32(j)

            @pl.when(j < C)
            def _():
                base = off(j)
                pltpu.sync_copy(s_h.at[pl.ds(base, G_B)], si[slot])
                pltpu.sync_copy(r_h.at[pl.ds(base, G_B)], ri[slot])
                pltpu.async_copy(t1_h.at[si[slot]], rb[slot], sa[slot])

        def mid(j, slot):
            j = jnp.int32(j)

            @pl.when(j < C)
            def _():
                pltpu.make_async_copy(t1_h.at[si[slot]], rb[slot],
                                      sa[slot]).wait()
                pltpu.async_copy(t2_h.at[ri[slot]], rb[slot], sb[slot],
                                 add=True)

        def fin(j, slot):
            j = jnp.int32(j)

            @pl.when(j < C)
            def _():
                pltpu.make_async_copy(t2_h.at[ri[slot]], rb[slot],
                                      sb[slot]).wait()
                pltpu.sync_copy(rb[slot], o_h.at[pl.ds(off(j), G_B)])

        start(0, 0)
        start(1, 1)
        mid(0, 0)
        start(2, 2)
        mid(1, 1)

        def body(t, carry):
            j0 = 3 * t
            fin(j0, 0)
            start(j0 + 3, 0)
            mid(j0 + 2, 2)
            fin(j0 + 1, 1)
            start(j0 + 4, 1)
            mid(j0 + 3, 0)
            fin(j0 + 2, 2)
            start(j0 + 5, 2)
            mid(j0 + 4, 1)
            return carry

        lax.fori_loop(0, (C + 2) // 3, body, 0)

    return k(t1, t2, sidx, ridx)


HN = N // NC
TRASH = HN
ACC_R = HN + 8
WB_A = 312
WB_B = HN - 15 * WB_A


S_B = 128


def _scatter_add(vals, ridx):
    EQ = ridx.shape[0]
    per_s = EQ // NS
    s_full = per_s // S_B
    s_tail = per_s - s_full * S_B

    @functools.partial(
        pl.kernel,
        out_type=jax.ShapeDtypeStruct((N, H), jnp.float32),
        mesh=_mesh,
        scratch_types=[
            pltpu.VMEM((S_B,), jnp.int32),
            pltpu.VMEM((S_B, H), jnp.float32),
            pltpu.VMEM((S_B,), jnp.int32),
            pltpu.VMEM((S_B, H), jnp.float32),
            pltpu.VMEM((16,), jnp.int32),
            pltpu.VMEM((16, H), jnp.float32),
            pltpu.VMEM((WB_B, H), jnp.float32),
            pltpu.VMEM_SHARED((ACC_R, H), jnp.float32),
            pltpu.SemaphoreType.DMA,
            pltpu.SemaphoreType.DMA,
        ],
    )
    def k(v_h, r_h, o_h, idx0, rows0, idx1, rows1, idxt, rowst, zbuf, acc,
          sm0, sm1):
        c = lax.axis_index("c")
        s = lax.axis_index("s")
        lo = c * HN

        def zrow(i, carry):
            def zcol(j, cc):
                zbuf[i, pl.ds(j * 16, 16)] = jnp.zeros((16,), jnp.float32)
                return cc
            return lax.fori_loop(0, H // 16, zcol, carry)

        lax.fori_loop(0, WB_B, zrow, 0)

        @pl.when(s < 15)
        def _():
            pltpu.sync_copy(zbuf.at[pl.ds(0, WB_A)],
                            acc.at[pl.ds(s * WB_A, WB_A)])

        @pl.when(s == 15)
        def _():
            pltpu.sync_copy(zbuf, acc.at[pl.ds(15 * WB_A, WB_B)])

        plsc.subcore_barrier()

        idx = (idx0, idx1)
        rows = (rows0, rows1)
        sm = (sm0, sm1)

        def rebase(ref, nvec):
            for t in range(nvec):
                v = ref[pl.ds(t * 16, 16)] - lo
                ok = (v >= 0) & (v < HN)
                ref[pl.ds(t * 16, 16)] = jnp.where(ok, v, TRASH)

        def start(j, slot):
            base = s * per_s + j * S_B
            pltpu.async_copy(v_h.at[pl.ds(base, S_B)], rows[slot], sm[slot])
            pltpu.sync_copy(r_h.at[pl.ds(base, S_B)], idx[slot])

        def finish(j, slot):
            base = s * per_s + j * S_B
            pltpu.make_async_copy(v_h.at[pl.ds(base, S_B)], rows[slot],
                                  sm[slot]).wait()
            rebase(idx[slot], S_B // 16)
            pltpu.sync_copy(rows[slot], acc.at[idx[slot]], add=True)

        _pipe(s_full, start, finish)
        if s_tail:
            tbase = s * per_s + s_full * S_B
            pltpu.sync_copy(v_h.at[pl.ds(tbase, s_tail)], rowst)
            pltpu.sync_copy(r_h.at[pl.ds(tbase, s_tail)], idxt)
            rebase(idxt, s_tail // 16)
            pltpu.sync_copy(rowst, acc.at[idxt], add=True)
        plsc.subcore_barrier()

        @pl.when(s < 15)
        def _():
            pltpu.sync_copy(acc.at[pl.ds(s * WB_A, WB_A)],
                            o_h.at[pl.ds(lo + s * WB_A, WB_A)])

        @pl.when(s == 15)
        def _():
            pltpu.sync_copy(acc.at[pl.ds(15 * WB_A, WB_B)],
                            o_h.at[pl.ds(lo + 15 * WB_A, WB_B)])

    return k(vals, ridx)


def _ln(h, g, b):
    m = jnp.mean(h, axis=-1, keepdims=True)
    v = jnp.mean((h - m) * (h - m), axis=-1, keepdims=True)
    return (h - m) * lax.rsqrt(v + 1e-5) * g + b


def _dot(a, b):
    return jnp.dot(a, b, preferred_element_type=jnp.float32)


def _full(shape):
    return pl.BlockSpec(shape, lambda i: (0,) * len(shape))


def _rows(blk, d):
    return pl.BlockSpec((blk, d), lambda i: (i, 0))


N_BLK = 2000
E_BLK = 2000


def _tc_enc_node(x16, w0, b0, w1, b1, g, bl, wa, wb):
    def body(x_r, w0_r, b0_r, w1_r, b1_r, g_r, bl_r, wa_r, wb_r,
             xh_r, a1_r, a2_r):
        h = jnp.maximum(_dot(x_r[...], w0_r[...]) + b0_r[...], 0.0)
        xh = _ln(_dot(h, w1_r[...]) + b1_r[...], g_r[...], bl_r[...])
        xh_r[...] = xh
        a1_r[...] = _dot(xh, wa_r[...])
        a2_r[...] = _dot(xh, wb_r[...])

    o = jax.ShapeDtypeStruct((N, H), jnp.float32)
    return pl.pallas_call(
        body,
        grid=(N // N_BLK,),
        in_specs=[_rows(N_BLK, 16), _full((16, H)), _full((1, H)),
                  _full((H, H)), _full((1, H)), _full((1, H)), _full((1, H)),
                  _full((H, H)), _full((H, H))],
        out_specs=[_rows(N_BLK, H)] * 3,
        out_shape=[o, o, o],
    )(x16, w0, b0, w1, b1, g, bl, wa, wb)


def _tc_enc_edge(rel_in, wlin, wd, wdw, b0, w1, b1, g, bl):
    def body(rel_r, wlin_r, wd_r, wdw_r, b0_r, w1_r, b1_r, g_r, bl_r,
             eh_r):
        rel = rel_r[...]
        d2 = rel[:, 0:1] * rel[:, 0:1] + rel[:, 1:2] * rel[:, 1:2]
        dw2 = rel[:, 2:3] * rel[:, 2:3] + rel[:, 3:4] * rel[:, 3:4]
        pre = (_dot(rel, wlin_r[...])
               + jnp.sqrt(d2) * wd_r[...]
               + jnp.sqrt(dw2) * wdw_r[...] + b0_r[...])
        h = jnp.maximum(pre, 0.0)
        eh_r[...] = _ln(_dot(h, w1_r[...]) + b1_r[...], g_r[...], bl_r[...])

    EQ = rel_in.shape[0]
    return pl.pallas_call(
        body,
        grid=(EQ // E_BLK,),
        in_specs=[_rows(E_BLK, H), _full((H, H)),
                  _full((1, H)), _full((1, H)), _full((1, H)),
                  _full((H, H)), _full((1, H)), _full((1, H)), _full((1, H))],
        out_specs=[_rows(E_BLK, H)],
        out_shape=[jax.ShapeDtypeStruct((EQ, H), jnp.float32)],
    )(rel_in, wlin, wd, wdw, b0, w1, b1, g, bl)[0]


def _tc_edge_step(gsum, eh, w3, b0, w1, b1, g, bl, want_resid=True):
    def body(gs_r, eh_r, w3_r, b0_r, w1_r, b1_r, g_r, bl_r,
             ne_r, *rest):
        eh_v = eh_r[...]
        t = jnp.maximum(gs_r[...] + _dot(eh_v, w3_r[...])
                        + b0_r[...], 0.0)
        t = jnp.maximum(_dot(t, w1_r[...]) + b1_r[...], 0.0)
        ne = _ln(t, g_r[...], bl_r[...])
        ne_r[...] = ne
        if rest:
            rest[0][...] = ne + eh_v

    EQ = gsum.shape[0]
    o = jax.ShapeDtypeStruct((EQ, H), jnp.float32)
    n_out = 2 if want_resid else 1
    res = pl.pallas_call(
        body,
        grid=(EQ // E_BLK,),
        in_specs=[_rows(E_BLK, H)] * 2 + [_full((H, H)), _full((1, H)),
                  _full((H, H)), _full((1, H)), _full((1, H)), _full((1, H))],
        out_specs=[_rows(E_BLK, H)] * n_out,
        out_shape=[o] * n_out,
    )(gsum, eh, w3, b0, w1, b1, g, bl)
    return res if want_resid else (res[0], None)


def _tc_node_step(xh, p0, p1, w0a, w0b, b0, w1, b1, g, bl, wa, wb):
    def body(xh_r, p0_r, p1_r, w0a_r, w0b_r, b0_r, w1_r, b1_r, g_r, bl_r,
             wa_r, wb_r, xn_r, a1_r, a2_r):
        xh_v = xh_r[...]
        ag = p0_r[...] + p1_r[...]
        t = jnp.maximum(_dot(xh_v, w0a_r[...]) + _dot(ag, w0b_r[...])
                        + b0_r[...], 0.0)
        t = jnp.maximum(_dot(t, w1_r[...]) + b1_r[...], 0.0)
        xn = _ln(t, g_r[...], bl_r[...]) + xh_v
        xn_r[...] = xn
        a1_r[...] = _dot(xn, wa_r[...])
        a2_r[...] = _dot(xn, wb_r[...])

    o = jax.ShapeDtypeStruct((N, H), jnp.float32)
    return pl.pallas_call(
        body,
        grid=(N // N_BLK,),
        in_specs=[_rows(N_BLK, H)] * 3 + [_full((H, H)), _full((H, H)),
                  _full((1, H)), _full((H, H)), _full((1, H)), _full((1, H)),
                  _full((1, H)), _full((H, H)), _full((H, H))],
        out_specs=[_rows(N_BLK, H)] * 3,
        out_shape=[o, o, o],
    )(xh, p0, p1, w0a, w0b, b0, w1, b1, g, bl, wa, wb)


def _tc_node_last(xh, p0, p1, w0a, w0b, b0, w1, b1, g, bl,
                  dw0, dbw0, wdw8, dp0, dbp0, wdp8, bd8):
    def body(xh_r, p0_r, p1_r, w0a_r, w0b_r, b0_r, w1_r, b1_r, g_r, bl_r,
             dw0_r, dbw0_r, wdw8_r, dp0_r, dbp0_r, wdp8_r, bd8_r, o_r):
        xh_v = xh_r[...]
        ag = p0_r[...] + p1_r[...]
        t = jnp.maximum(_dot(xh_v, w0a_r[...]) + _dot(ag, w0b_r[...])
                        + b0_r[...], 0.0)
        t = jnp.maximum(_dot(t, w1_r[...]) + b1_r[...], 0.0)
        xn = _ln(t, g_r[...], bl_r[...]) + xh_v
        d1 = jnp.maximum(_dot(xn, dw0_r[...]) + dbw0_r[...], 0.0)
        d2 = jnp.maximum(_dot(xn, dp0_r[...]) + dbp0_r[...], 0.0)
        o_r[...] = _dot(d1, wdw8_r[...]) + _dot(d2, wdp8_r[...]) + bd8_r[...]

    return pl.pallas_call(
        body,
        grid=(N // N_BLK,),
        in_specs=[_rows(N_BLK, H)] * 3 + [_full((H, H)), _full((H, H)),
                  _full((1, H)), _full((H, H)), _full((1, H)), _full((1, H)),
                  _full((1, H)), _full((H, H)), _full((1, H)), _full((H, 8)),
                  _full((H, H)), _full((1, H)), _full((H, 8)), _full((1, 8))],
        out_specs=[_rows(N_BLK, 8)],
        out_shape=[jax.ShapeDtypeStruct((N, 8), jnp.float32)],
    )(xh, p0, p1, w0a, w0b, b0, w1, b1, g, bl,
      dw0, dbw0, wdw8, dp0, dbp0, wdp8, bd8)[0]


def kernel(world_pos, mesh_pos, prev_world_pos, phi, prev_phi, swelling_phi,
           swelling_phi_rate, swelling_phi_rate_prev, node_type, mat_param,
           edge_index, params):
    f32 = jnp.float32
    senders = edge_index[0].astype(jnp.int32)
    receivers = edge_index[1].astype(jnp.int32)

    x16 = jnp.concatenate(
        [phi, prev_phi, swelling_phi, swelling_phi_rate,
         swelling_phi_rate_prev, node_type,
         jnp.zeros((N, 2), f32)], axis=1)
    ne0w = params["ne0"]["w"]
    w0p = jnp.concatenate(
        [(ne0w[0] + ne0w[1])[None], (-ne0w[1])[None], ne0w[2:],
         jnp.zeros((2, H), f32)], axis=0)

    P = jnp.concatenate([mesh_pos, world_pos, phi, jnp.zeros((N, H - 5), f32)],
                        axis=1)
    ee0w = params["ee0"]["w"]
    wlin = jnp.concatenate([ee0w[0:2], ee0w[3:5], ee0w[6:7],
                            jnp.zeros((H - 5, H), f32)], axis=0)
    wd = ee0w[2][None]
    wdw = ee0w[5][None]

    def r1(v):
        return v[None]

    pr0 = params["procs"][0]
    x_h, a1, a2 = _tc_enc_node(
        x16, w0p, r1(params["ne0"]["b"]), params["ne1"]["w"],
        r1(params["ne1"]["b"]), r1(params["ne_ln"]["g"]),
        r1(params["ne_ln"]["b"]),
        pr0["e0"]["w"][0:H], pr0["e0"]["w"][H:2 * H])

    E2 = E // 2
    sid = (senders[:E2], senders[E2:])
    rid = (receivers[:E2], receivers[E2:])

    negP = -P
    e_h = []
    for q in range(2):
        rel = _gather_sum(P, negP, sid[q], rid[q])
        e_h.append(_tc_enc_edge(
            rel, wlin, wd, wdw, r1(params["ee0"]["b"]),
            params["ee1"]["w"], r1(params["ee1"]["b"]),
            r1(params["ee_ln"]["g"]), r1(params["ee_ln"]["b"])))

    dec = None
    for k in range(3):
        pr = params["procs"][k]
        part = []
        for q in range(2):
            gsum = _gather_sum(a1, a2, sid[q], rid[q])
            new_e, e_h[q] = _tc_edge_step(
                gsum, e_h[q], pr["e0"]["w"][2 * H:3 * H], r1(pr["e0"]["b"]),
                pr["e1"]["w"], r1(pr["e1"]["b"]), r1(pr["e_ln"]["g"]),
                r1(pr["e_ln"]["b"]), want_resid=(k < 2))
            part.append(_scatter_add(new_e, rid[q]))
        nargs = (x_h, part[0], part[1], pr["n0"]["w"][0:H],
                 pr["n0"]["w"][H:2 * H],
                 r1(pr["n0"]["b"]), pr["n1"]["w"], r1(pr["n1"]["b"]),
                 r1(pr["n_ln"]["g"]), r1(pr["n_ln"]["b"]))
        if k < 2:
            prn = params["procs"][k + 1]
            x_h, a1, a2 = _tc_node_step(
                *nargs, prn["e0"]["w"][0:H], prn["e0"]["w"][H:2 * H])
        else:
            wdw8 = jnp.zeros((H, 8), f32).at[:, 0:2].set(params["dw1"]["w"])
            wdp8 = jnp.zeros((H, 8), f32).at[:, 2:3].set(params["dp1"]["w"])
            bd8 = jnp.zeros((1, 8), f32).at[0, 0:2].set(
                params["dw1"]["b"]).at[0, 2].set(params["dp1"]["b"][0])
            dec = _tc_node_last(
                *nargs, params["dw0"]["w"], r1(params["dw0"]["b"]), wdw8,
                params["dp0"]["w"], r1(params["dp0"]["b"]), wdp8, bd8)

    return dec[:, :3]


# speedup vs baseline: 7.7492x; 1.0240x over previous
import functools

import jax
import jax.numpy as jnp
from jax import lax
from jax.experimental import pallas as pl
from jax.experimental.pallas import tpu as pltpu
from jax.experimental.pallas import tpu_sc as plsc

N = 10000
E = 320000
H = 128

NC = 2
NS = 16
NW = NC * NS
SC_B = 80
PER_W = E // NW
SC_ITERS = PER_W // SC_B
ROW_A = 624
ROW_B = N - 15 * ROW_A

_mesh = plsc.VectorSubcoreMesh(core_axis_name="c", subcore_axis_name="s")


G_B = 128


def _pipe(C, start, finish):
    start(0, 0)

    def body(t, carry):
        j0 = 2 * t
        start(j0 + 1, 1)
        finish(j0, 0)
        start(j0 + 2, 0)
        finish(j0 + 1, 1)
        return carry

    if C % 2 == 1:
        lax.fori_loop(0, (C - 1) // 2, body, 0)
        finish(C - 1, 0)
    else:
        lax.fori_loop(0, (C - 2) // 2, body, 0)
        start(C - 1, 1)
        finish(C - 2, 0)
        finish(C - 1, 1)


def _gather_sum(t1, t2, sidx2, ridx2, EQ):
    D = t1.shape[1]
    R = EQ // G_B
    RB = R // NW
    REM = R - RB * NW
    C = RB + 1
    PRE = -(-(RB + 9) // 8) * 8
    out = jax.ShapeDtypeStruct((EQ, D), jnp.float32)

    @functools.partial(
        pl.kernel,
        out_type=out,
        mesh=_mesh,
        scratch_types=[
            pltpu.VMEM((PRE, G_B), jnp.int32),
            pltpu.VMEM((PRE, G_B), jnp.int32),
            pltpu.VMEM((G_B, D), jnp.float32),
            pltpu.VMEM((G_B, D), jnp.float32),
            pltpu.VMEM((G_B, D), jnp.float32),
            pltpu.SemaphoreType.DMA,
            pltpu.SemaphoreType.DMA,
            pltpu.SemaphoreType.DMA,
            pltpu.SemaphoreType.DMA,
            pltpu.SemaphoreType.DMA,
            pltpu.SemaphoreType.DMA,
        ],
    )
    def k(t1_h, t2_h, s_h, r_h, o_h, sbuf, rbuf, b0, b1, b2,
          sa0, sb0, sa1, sb1, sa2, sb2):
        wid = lax.axis_index("s") * NC + lax.axis_index("c")
        row0 = wid * RB + jnp.minimum(wid, REM)
        nr = RB + (wid < REM).astype(jnp.int32)
        row0a = (row0 // 8) * 8
        sk = row0 - row0a
        pltpu.sync_copy(s_h.at[pl.ds(row0a, PRE)], sbuf)
        pltpu.sync_copy(r_h.at[pl.ds(row0a, PRE)], rbuf)
        rb = (b0, b1, b2)
        sa = (sa0, sa1, sa2)
        sb = (sb0, sb1, sb2)

        def start(j, slot):
            j = jnp.int32(j)

            @pl.when(j < nr)
            def _():
                pltpu.async_copy(t1_h.at[sbuf.at[sk + j]], rb[slot], sa[slot])

        def mid(j, slot):
            j = jnp.int32(j)

            @pl.when(j < nr)
            def _():
                pltpu.make_async_copy(t1_h.at[sbuf.at[sk + j]], rb[slot],
                                      sa[slot]).wait()
                pltpu.async_copy(t2_h.at[rbuf.at[sk + j]], rb[slot], sb[slot],
                                 add=True)

        def fin(j, slot):
            j = jnp.int32(j)

            @pl.when(j < nr)
            def _():
                pltpu.make_async_copy(t2_h.at[rbuf.at[sk + j]], rb[slot],
                                      sb[slot]).wait()
                pltpu.sync_copy(rb[slot],
                                o_h.at[pl.ds((row0 + j) * G_B, G_B)])

        start(0, 0)
        start(1, 1)
        mid(0, 0)
        start(2, 2)
        mid(1, 1)

        def body(t, carry):
            j0 = 3 * t
            fin(j0, 0)
            start(j0 + 3, 0)
            mid(j0 + 2, 2)
            fin(j0 + 1, 1)
            start(j0 + 4, 1)
            mid(j0 + 3, 0)
            fin(j0 + 2, 2)
            start(j0 + 5, 2)
            mid(j0 + 4, 1)
            return carry

        lax.fori_loop(0, (C + 2) // 3, body, 0)

    return k(t1, t2, sidx2, ridx2)


HN = N // NC
TRASH = HN
ACC_R = HN + 8
WB_A = 312
WB_B = HN - 15 * WB_A


def _scatter_add(vals, ridx2, EQ):
    R = EQ // G_B
    RB = R // NS
    REM = R - RB * NS
    C = RB + 1
    PRE = -(-(RB + 9) // 8) * 8

    @functools.partial(
        pl.kernel,
        out_type=jax.ShapeDtypeStruct((N, H), jnp.float32),
        mesh=_mesh,
        scratch_types=[
            pltpu.VMEM((PRE, G_B), jnp.int32),
            pltpu.VMEM((G_B, H), jnp.float32),
            pltpu.VMEM((G_B, H), jnp.float32),
            pltpu.VMEM((WB_B, H), jnp.float32),
            pltpu.VMEM_SHARED((ACC_R, H), jnp.float32),
            pltpu.SemaphoreType.DMA,
            pltpu.SemaphoreType.DMA,
        ],
    )
    def k(v_h, r_h, o_h, ibuf, rows0, rows1, zbuf, acc, sm0, sm1):
        c = lax.axis_index("c")
        s = lax.axis_index("s")
        lo = c * HN
        row0 = s * RB + jnp.minimum(s, REM)
        nr = RB + (s < REM).astype(jnp.int32)
        row0a = (row0 // 8) * 8
        sk = row0 - row0a
        pltpu.sync_copy(r_h.at[pl.ds(row0a, PRE)], ibuf)

        def zrow(i, carry):
            def zcol(j, cc):
                zbuf[i, pl.ds(j * 16, 16)] = jnp.zeros((16,), jnp.float32)
                return cc
            return lax.fori_loop(0, H // 16, zcol, carry)

        lax.fori_loop(0, WB_B, zrow, 0)

        def brow(i, carry):
            def bcol(t, cc):
                v = ibuf[i, pl.ds(t * 16, 16)] - lo
                ok = (v >= 0) & (v < HN)
                ibuf[i, pl.ds(t * 16, 16)] = jnp.where(ok, v, TRASH)
                return cc
            return lax.fori_loop(0, G_B // 16, bcol, carry)

        lax.fori_loop(0, PRE, brow, 0)

        @pl.when(s < 15)
        def _():
            pltpu.sync_copy(zbuf.at[pl.ds(0, WB_A)],
                            acc.at[pl.ds(s * WB_A, WB_A)])

        @pl.when(s == 15)
        def _():
            pltpu.sync_copy(zbuf, acc.at[pl.ds(15 * WB_A, WB_B)])

        plsc.subcore_barrier()

        rows = (rows0, rows1)
        sm = (sm0, sm1)

        def start(j, slot):
            j = jnp.int32(j)

            @pl.when(j < nr)
            def _():
                pltpu.async_copy(v_h.at[pl.ds((row0 + j) * G_B, G_B)],
                                 rows[slot], sm[slot])

        def finish(j, slot):
            j = jnp.int32(j)

            @pl.when(j < nr)
            def _():
                pltpu.make_async_copy(
                    v_h.at[pl.ds((row0 + j) * G_B, G_B)],
                    rows[slot], sm[slot]).wait()
                pltpu.sync_copy(rows[slot], acc.at[ibuf.at[sk + j]],
                                add=True)

        start(0, 0)

        def body(t, carry):
            j0 = 2 * t
            start(j0 + 1, 1)
            finish(j0, 0)
            start(j0 + 2, 0)
            finish(j0 + 1, 1)
            return carry

        lax.fori_loop(0, (C + 1) // 2, body, 0)
        plsc.subcore_barrier()

        @pl.when(s < 15)
        def _():
            pltpu.sync_copy(acc.at[pl.ds(s * WB_A, WB_A)],
                            o_h.at[pl.ds(lo + s * WB_A, WB_A)])

        @pl.when(s == 15)
        def _():
            pltpu.sync_copy(acc.at[pl.ds(15 * WB_A, WB_B)],
                            o_h.at[pl.ds(lo + 15 * WB_A, WB_B)])

    return k(vals, ridx2)


def _ln(h, g, b):
    m = jnp.mean(h, axis=-1, keepdims=True)
    v = jnp.mean((h - m) * (h - m), axis=-1, keepdims=True)
    return (h - m) * lax.rsqrt(v + 1e-5) * g + b


def _dot(a, b):
    return jnp.dot(a, b, preferred_element_type=jnp.float32)


def _full(shape):
    return pl.BlockSpec(shape, lambda i: (0,) * len(shape))


def _rows(blk, d):
    return pl.BlockSpec((blk, d), lambda i: (i, 0))


N_BLK = 2000
E_BLK = 2000


def _tc_enc_node(x16, w0, b0, w1, b1, g, bl, wa, wb):
    def body(x_r, w0_r, b0_r, w1_r, b1_r, g_r, bl_r, wa_r, wb_r,
             xh_r, a1_r, a2_r):
        h = jnp.maximum(_dot(x_r[...], w0_r[...]) + b0_r[...], 0.0)
        xh = _ln(_dot(h, w1_r[...]) + b1_r[...], g_r[...], bl_r[...])
        xh_r[...] = xh
        a1_r[...] = _dot(xh, wa_r[...])
        a2_r[...] = _dot(xh, wb_r[...])

    o = jax.ShapeDtypeStruct((N, H), jnp.float32)
    return pl.pallas_call(
        body,
        grid=(N // N_BLK,),
        in_specs=[_rows(N_BLK, 16), _full((16, H)), _full((1, H)),
                  _full((H, H)), _full((1, H)), _full((1, H)), _full((1, H)),
                  _full((H, H)), _full((H, H))],
        out_specs=[_rows(N_BLK, H)] * 3,
        out_shape=[o, o, o],
    )(x16, w0, b0, w1, b1, g, bl, wa, wb)


def _tc_enc_edge(rel_in, wlin, wd, wdw, b0, w1, b1, g, bl):
    def body(rel_r, wlin_r, wd_r, wdw_r, b0_r, w1_r, b1_r, g_r, bl_r,
             eh_r):
        rel = rel_r[...]
        d2 = rel[:, 0:1] * rel[:, 0:1] + rel[:, 1:2] * rel[:, 1:2]
        dw2 = rel[:, 2:3] * rel[:, 2:3] + rel[:, 3:4] * rel[:, 3:4]
        pre = (_dot(rel, wlin_r[...])
               + jnp.sqrt(d2) * wd_r[...]
               + jnp.sqrt(dw2) * wdw_r[...] + b0_r[...])
        h = jnp.maximum(pre, 0.0)
        eh_r[...] = _ln(_dot(h, w1_r[...]) + b1_r[...], g_r[...], bl_r[...])

    EQ = rel_in.shape[0]
    return pl.pallas_call(
        body,
        grid=(EQ // E_BLK,),
        in_specs=[_rows(E_BLK, H), _full((H, H)),
                  _full((1, H)), _full((1, H)), _full((1, H)),
                  _full((H, H)), _full((1, H)), _full((1, H)), _full((1, H))],
        out_specs=[_rows(E_BLK, H)],
        out_shape=[jax.ShapeDtypeStruct((EQ, H), jnp.float32)],
    )(rel_in, wlin, wd, wdw, b0, w1, b1, g, bl)[0]


def _tc_edge_step(gsum, eh, w3, b0, w1, b1, g, bl, want_resid=True):
    def body(gs_r, eh_r, w3_r, b0_r, w1_r, b1_r, g_r, bl_r,
             ne_r, *rest):
        eh_v = eh_r[...]
        t = jnp.maximum(gs_r[...] + _dot(eh_v, w3_r[...])
                        + b0_r[...], 0.0)
        t = jnp.maximum(_dot(t, w1_r[...]) + b1_r[...], 0.0)
        ne = _ln(t, g_r[...], bl_r[...])
        ne_r[...] = ne
        if rest:
            rest[0][...] = ne + eh_v

    EQ = gsum.shape[0]
    o = jax.ShapeDtypeStruct((EQ, H), jnp.float32)
    n_out = 2 if want_resid else 1
    res = pl.pallas_call(
        body,
        grid=(EQ // E_BLK,),
        in_specs=[_rows(E_BLK, H)] * 2 + [_full((H, H)), _full((1, H)),
                  _full((H, H)), _full((1, H)), _full((1, H)), _full((1, H))],
        out_specs=[_rows(E_BLK, H)] * n_out,
        out_shape=[o] * n_out,
    )(gsum, eh, w3, b0, w1, b1, g, bl)
    return res if want_resid else (res[0], None)


def _tc_node_step(xh, p0, p1, w0a, w0b, b0, w1, b1, g, bl, wa, wb):
    def body(xh_r, p0_r, p1_r, w0a_r, w0b_r, b0_r, w1_r, b1_r, g_r, bl_r,
             wa_r, wb_r, xn_r, a1_r, a2_r):
        xh_v = xh_r[...]
        ag = p0_r[...] + p1_r[...]
        t = jnp.maximum(_dot(xh_v, w0a_r[...]) + _dot(ag, w0b_r[...])
                        + b0_r[...], 0.0)
        t = jnp.maximum(_dot(t, w1_r[...]) + b1_r[...], 0.0)
        xn = _ln(t, g_r[...], bl_r[...]) + xh_v
        xn_r[...] = xn
        a1_r[...] = _dot(xn, wa_r[...])
        a2_r[...] = _dot(xn, wb_r[...])

    o = jax.ShapeDtypeStruct((N, H), jnp.float32)
    return pl.pallas_call(
        body,
        grid=(N // N_BLK,),
        in_specs=[_rows(N_BLK, H)] * 3 + [_full((H, H)), _full((H, H)),
                  _full((1, H)), _full((H, H)), _full((1, H)), _full((1, H)),
                  _full((1, H)), _full((H, H)), _full((H, H))],
        out_specs=[_rows(N_BLK, H)] * 3,
        out_shape=[o, o, o],
    )(xh, p0, p1, w0a, w0b, b0, w1, b1, g, bl, wa, wb)


def _tc_node_last(xh, p0, p1, w0a, w0b, b0, w1, b1, g, bl,
                  dw0, dbw0, wdw8, dp0, dbp0, wdp8, bd8):
    def body(xh_r, p0_r, p1_r, w0a_r, w0b_r, b0_r, w1_r, b1_r, g_r, bl_r,
             dw0_r, dbw0_r, wdw8_r, dp0_r, dbp0_r, wdp8_r, bd8_r, o_r):
        xh_v = xh_r[...]
        ag = p0_r[...] + p1_r[...]
        t = jnp.maximum(_dot(xh_v, w0a_r[...]) + _dot(ag, w0b_r[...])
                        + b0_r[...], 0.0)
        t = jnp.maximum(_dot(t, w1_r[...]) + b1_r[...], 0.0)
        xn = _ln(t, g_r[...], bl_r[...]) + xh_v
        d1 = jnp.maximum(_dot(xn, dw0_r[...]) + dbw0_r[...], 0.0)
        d2 = jnp.maximum(_dot(xn, dp0_r[...]) + dbp0_r[...], 0.0)
        o_r[...] = _dot(d1, wdw8_r[...]) + _dot(d2, wdp8_r[...]) + bd8_r[...]

    return pl.pallas_call(
        body,
        grid=(N // N_BLK,),
        in_specs=[_rows(N_BLK, H)] * 3 + [_full((H, H)), _full((H, H)),
                  _full((1, H)), _full((H, H)), _full((1, H)), _full((1, H)),
                  _full((1, H)), _full((H, H)), _full((1, H)), _full((H, 8)),
                  _full((H, H)), _full((1, H)), _full((H, 8)), _full((1, 8))],
        out_specs=[_rows(N_BLK, 8)],
        out_shape=[jax.ShapeDtypeStruct((N, 8), jnp.float32)],
    )(xh, p0, p1, w0a, w0b, b0, w1, b1, g, bl,
      dw0, dbw0, wdw8, dp0, dbp0, wdp8, bd8)[0]


def kernel(world_pos, mesh_pos, prev_world_pos, phi, prev_phi, swelling_phi,
           swelling_phi_rate, swelling_phi_rate_prev, node_type, mat_param,
           edge_index, params):
    f32 = jnp.float32
    senders = edge_index[0].astype(jnp.int32)
    receivers = edge_index[1].astype(jnp.int32)

    x16 = jnp.concatenate(
        [phi, prev_phi, swelling_phi, swelling_phi_rate,
         swelling_phi_rate_prev, node_type,
         jnp.zeros((N, 2), f32)], axis=1)
    ne0w = params["ne0"]["w"]
    w0p = jnp.concatenate(
        [(ne0w[0] + ne0w[1])[None], (-ne0w[1])[None], ne0w[2:],
         jnp.zeros((2, H), f32)], axis=0)

    P = jnp.concatenate([mesh_pos, world_pos, phi, jnp.zeros((N, H - 5), f32)],
                        axis=1)
    ee0w = params["ee0"]["w"]
    wlin = jnp.concatenate([ee0w[0:2], ee0w[3:5], ee0w[6:7],
                            jnp.zeros((H - 5, H), f32)], axis=0)
    wd = ee0w[2][None]
    wdw = ee0w[5][None]

    def r1(v):
        return v[None]

    pr0 = params["procs"][0]
    x_h, a1, a2 = _tc_enc_node(
        x16, w0p, r1(params["ne0"]["b"]), params["ne1"]["w"],
        r1(params["ne1"]["b"]), r1(params["ne_ln"]["g"]),
        r1(params["ne_ln"]["b"]),
        pr0["e0"]["w"][0:H], pr0["e0"]["w"][H:2 * H])

    E2 = E // 2
    RQ = E2 // G_B
    s2 = senders.reshape(E // G_B, G_B)
    r2 = receivers.reshape(E // G_B, G_B)
    padz = jnp.zeros((8, G_B), jnp.int32)
    sid = tuple(jnp.concatenate([s2[q * RQ:(q + 1) * RQ], padz])
                for q in range(2))
    rid = tuple(jnp.concatenate([r2[q * RQ:(q + 1) * RQ], padz])
                for q in range(2))

    negP = -P
    e_h = []
    for q in range(2):
        rel = _gather_sum(P, negP, sid[q], rid[q], E2)
        e_h.append(_tc_enc_edge(
            rel, wlin, wd, wdw, r1(params["ee0"]["b"]),
            params["ee1"]["w"], r1(params["ee1"]["b"]),
            r1(params["ee_ln"]["g"]), r1(params["ee_ln"]["b"])))

    dec = None
    for k in range(3):
        pr = params["procs"][k]
        part = []
        for q in range(2):
            gsum = _gather_sum(a1, a2, sid[q], rid[q], E2)
            new_e, e_h[q] = _tc_edge_step(
                gsum, e_h[q], pr["e0"]["w"][2 * H:3 * H], r1(pr["e0"]["b"]),
                pr["e1"]["w"], r1(pr["e1"]["b"]), r1(pr["e_ln"]["g"]),
                r1(pr["e_ln"]["b"]), want_resid=(k < 2))
            part.append(_scatter_add(new_e, rid[q], E2))
        nargs = (x_h, part[0], part[1], pr["n0"]["w"][0:H],
                 pr["n0"]["w"][H:2 * H],
                 r1(pr["n0"]["b"]), pr["n1"]["w"], r1(pr["n1"]["b"]),
                 r1(pr["n_ln"]["g"]), r1(pr["n_ln"]["b"]))
        if k < 2:
            prn = params["procs"][k + 1]
            x_h, a1, a2 = _tc_node_step(
                *nargs, prn["e0"]["w"][0:H], prn["e0"]["w"][H:2 * H])
        else:
            wdw8 = jnp.zeros((H, 8), f32).at[:, 0:2].set(params["dw1"]["w"])
            wdp8 = jnp.zeros((H, 8), f32).at[:, 2:3].set(params["dp1"]["w"])
            bd8 = jnp.zeros((1, 8), f32).at[0, 0:2].set(
                params["dw1"]["b"]).at[0, 2].set(params["dp1"]["b"][0])
            dec = _tc_node_last(
                *nargs, params["dw0"]["w"], r1(params["dw0"]["b"]), wdw8,
                params["dp0"]["w"], r1(params["dp0"]["b"]), wdp8, bd8)

    return dec[:, :3]
